# baseline (speedup 1.0000x reference)
import functools

import jax
import jax.numpy as jnp
from jax import lax
from jax.experimental import pallas as pl
from jax.experimental.pallas import tpu as pltpu
from jax.experimental.pallas import tpu_sc as plsc

B = 16
P = 131072
RES = 50
NVOX = RES * RES * RES
GRID_PAD = 125008
OUT_ROW = NVOX * 4

NC, NS, L = 2, 16, 16

CH = 512
NECH = P // CH
XCHUNK = 512
XOUT = XCHUNK * 4
NXFULL = NVOX // XCHUNK
TAIL_GRID = NVOX - NXFULL * XCHUNK
TAIL_OUT = TAIL_GRID * 4

_mesh = plsc.VectorSubcoreMesh(
    core_axis_name="c", subcore_axis_name="s", num_cores=NC, num_subcores=NS)


@functools.partial(
    pl.kernel,
    out_type=jax.ShapeDtypeStruct((B * OUT_ROW,), jnp.float32),
    mesh=_mesh,
    compiler_params=pltpu.CompilerParams(needs_layout_passes=False),
    scratch_types=[
        pltpu.VMEM((GRID_PAD,), jnp.float32),
        pltpu.VMEM((4096,), jnp.float32),
        pltpu.SemaphoreType.DMA,
        pltpu.SemaphoreType.DMA,
        pltpu.SemaphoreType.DMA,
        pltpu.SemaphoreType.DMA,
    ],
)
def _voxelize(pts_hbm, vf_hbm, out_hbm, grid_v, buf_v,
              insem0, insem1, outsem0, outsem1):
    cid = lax.axis_index("c")
    sid = lax.axis_index("s")
    wid = sid * NC + cid

    @pl.when(wid < B)
    def _work():
        b = wid
        zeros16 = jnp.zeros((L,), jnp.float32)
        ones16 = jnp.ones((L,), jnp.float32)
        insems = (insem0, insem1)
        outsems = (outsem0, outsem1)

        def in_copies(c, bu):
            cps = []
            for ch in range(3):
                cps.append(pltpu.make_async_copy(
                    pts_hbm.at[ch, b, pl.ds(c * CH, CH)],
                    buf_v.at[pl.ds(ch * 2 * CH + bu * CH, CH)],
                    insems[bu]))
            cps.append(pltpu.make_async_copy(
                vf_hbm.at[b, pl.ds(c * CH, CH)],
                buf_v.at[pl.ds(3 * 2 * CH + bu * CH, CH)],
                insems[bu]))
            return cps

        def out_copy(c, bu):
            return pltpu.make_async_copy(
                buf_v.at[pl.ds(bu * XOUT, XOUT)],
                out_hbm.at[pl.ds(b * OUT_ROW + c * XOUT, XOUT)],
                outsems[bu])

        for cp in in_copies(0, 0):
            cp.start()
        for cp in in_copies(1, 1):
            cp.start()

        def zbody(i, carry):
            grid_v[pl.ds(i * L, L)] = zeros16
            return carry
        lax.fori_loop(0, GRID_PAD // L, zbody, 0, unroll=8)

        def scpair(g, carry):
            for bu in (0, 1):
                c = g * 2 + bu
                for cp in in_copies(c, bu):
                    cp.wait()

                def vbody(v, carry2, bu=bu):
                    def coord(ch, bu=bu):
                        p = buf_v[pl.ds(ch * 2 * CH + bu * CH + v * L, L)]
                        t = (p + 2.0) * 0.25 * 49.0
                        t = jnp.minimum(jnp.maximum(t, 0.0), 49.0)
                        return t.astype(jnp.int32)

                    i, j, k = coord(0), coord(1), coord(2)
                    lin = (i * 50 + j) * 50 + k
                    vf = buf_v[pl.ds(3 * 2 * CH + bu * CH + v * L, L)]
                    plsc.store_scatter(grid_v, [lin], ones16, mask=vf > 0.0)
                    return carry2
                lax.fori_loop(0, CH // L, vbody, 0, unroll=8)

                @pl.when(c + 2 < NECH)
                def _(c=c, bu=bu):
                    for cp in in_copies(c + 2, bu):
                        cp.start()
            return carry
        lax.fori_loop(0, NECH // 2, scpair, 0)

        def zb(i, carry):
            buf_v[pl.ds(i * L, L)] = zeros16
            return carry
        lax.fori_loop(0, 4096 // L, zb, 0, unroll=8)

        idx0 = lax.iota(jnp.int32, L) * 4

        def fill(c, bu, nvec):
            def fb(v, carry, bu=bu):
                vals = grid_v[pl.ds(c * XCHUNK + v * L, L)]
                plsc.store_scatter(
                    buf_v, [idx0 + (bu * XOUT + v * (L * 4))], vals)
                return carry
            lax.fori_loop(0, nvec, fb, 0, unroll=8)

        def xpair(g, carry):
            for bu in (0, 1):
                c = g * 2 + bu

                @pl.when(c >= 2)
                def _(c=c, bu=bu):
                    out_copy(c - 2, bu).wait()

                fill(c, bu, XCHUNK // L)
                out_copy(c, bu).start()
            return carry
        lax.fori_loop(0, NXFULL // 2, xpair, 0)

        out_copy(NXFULL - 2, 0).wait()
        fill(NXFULL, 0, (TAIL_GRID + L - 1) // L)
        tail = pltpu.make_async_copy(
            buf_v.at[pl.ds(0, TAIL_OUT)],
            out_hbm.at[pl.ds(b * OUT_ROW + NXFULL * XOUT, TAIL_OUT)],
            outsems[0])
        tail.start()
        out_copy(NXFULL - 1, 1).wait()
        tail.wait()


def kernel(pointclouds, valid_points):
    pts_t = pointclouds.transpose(2, 0, 1)
    vf = valid_points.astype(jnp.float32)
    flat = _voxelize(pts_t, vf)
    return flat.reshape(B, RES, RES, RES, 4)


# speedup vs baseline: 6.1170x; 6.1170x over previous
import functools

import jax
import jax.numpy as jnp
from jax import lax
from jax.experimental import pallas as pl
from jax.experimental.pallas import tpu as pltpu
from jax.experimental.pallas import tpu_sc as plsc

B = 16
P = 131072
RES = 50
NVOX = RES * RES * RES
GRID_PAD = 125008
OUT_ROW = NVOX * 4

NC, NS, L = 2, 16, 16

BB = 8


def _enc_body(pts_ref, valid_ref, enc_ref):
    def coord(c):
        t = (pts_ref[c] + 2.0) * 0.25 * 49.0
        ti = jnp.floor(t).astype(jnp.int32)
        return jnp.clip(ti, 0, 49)

    i, j, k = coord(0), coord(1), coord(2)
    lin = (i * 50 + j) * 50 + k
    enc_ref[...] = jnp.where(valid_ref[...], lin, -1).reshape(BB * P)


def _encode(pts_t, valid):
    return pl.pallas_call(
        _enc_body,
        grid=(B // BB,),
        in_specs=[
            pl.BlockSpec((3, BB, P), lambda b: (0, b, 0)),
            pl.BlockSpec((BB, P), lambda b: (b, 0)),
        ],
        out_specs=pl.BlockSpec((BB * P,), lambda b: (b,)),
        out_shape=jax.ShapeDtypeStruct((B * P,), jnp.int32),
    )(pts_t, valid)


ECHUNK = 2048
NECH = P // ECHUNK
OCHUNK = 2048
NOFULL = NVOX // OCHUNK
OTAIL = NVOX - NOFULL * OCHUNK

_mesh = plsc.VectorSubcoreMesh(
    core_axis_name="c", subcore_axis_name="s", num_cores=NC, num_subcores=NS)


ONE_F32_BITS = 0x3F800000


@functools.partial(
    pl.kernel,
    out_type=jax.ShapeDtypeStruct((B * NVOX,), jnp.float32),
    mesh=_mesh,
    compiler_params=pltpu.CompilerParams(needs_layout_passes=False),
    scratch_types=[
        pltpu.VMEM((GRID_PAD,), jnp.float32),
        pltpu.VMEM((2 * ECHUNK,), jnp.int32),
        pltpu.SemaphoreType.DMA,
        pltpu.SemaphoreType.DMA,
        pltpu.SemaphoreType.DMA,
    ],
)
def _voxelize(enc_hbm, out_hbm, grid_v, buf_v, insem0, insem1, outsem):
    cid = lax.axis_index("c")
    sid = lax.axis_index("s")
    wid = sid * NC + cid

    @pl.when(wid < B)
    def _work():
        b = wid
        zeros16 = jnp.zeros((L,), jnp.float32)
        ones16 = jnp.ones((L,), jnp.float32)
        insems = (insem0, insem1)

        def in_copy(c, bu):
            return pltpu.make_async_copy(
                enc_hbm.at[pl.ds(b * P + c * ECHUNK, ECHUNK)],
                buf_v.at[pl.ds(bu * ECHUNK, ECHUNK)],
                insems[bu])

        def out_copy(c, n):
            return pltpu.make_async_copy(
                grid_v.at[pl.ds(c * OCHUNK, n)],
                out_hbm.at[pl.ds(b * NVOX + c * OCHUNK, n)],
                outsem)

        in_copy(0, 0).start()
        in_copy(1, 1).start()

        def zbody(i, carry):
            grid_v[pl.ds(i * L, L)] = zeros16
            return carry
        lax.fori_loop(0, GRID_PAD // L, zbody, 0, unroll=8)

        def scpair(g, carry):
            for bu in (0, 1):
                c = g * 2 + bu
                in_copy(c, bu).wait()

                def vbody(v, carry2, bu=bu):
                    ev = buf_v[pl.ds(bu * ECHUNK + v * L, L)]
                    plsc.store_scatter(grid_v, [ev], ones16, mask=ev >= 0)
                    return carry2
                lax.fori_loop(0, ECHUNK // L, vbody, 0, unroll=8)

                @pl.when(c + 2 < NECH)
                def _(c=c, bu=bu):
                    in_copy(c + 2, bu).start()
            return carry
        lax.fori_loop(0, NECH // 2, scpair, 0)

        for c in range(NOFULL):
            out_copy(c, OCHUNK).start()
        out_copy(NOFULL, OTAIL).start()
        for c in range(NOFULL):
            out_copy(c, OCHUNK).wait()
        out_copy(NOFULL, OTAIL).wait()


def kernel(pointclouds, valid_points):
    pts_t = pointclouds.transpose(2, 0, 1)
    enc = _encode(pts_t, valid_points)
    occ = _voxelize(enc)
    occ5 = occ.reshape(B, RES, RES, RES, 1)
    return jnp.pad(occ5, ((0, 0), (0, 0), (0, 0), (0, 0), (0, 3)))


# speedup vs baseline: 6.1178x; 1.0001x over previous
import functools

import jax
import jax.numpy as jnp
from jax import lax
from jax.experimental import pallas as pl
from jax.experimental.pallas import tpu as pltpu
from jax.experimental.pallas import tpu_sc as plsc

B = 16
P = 131072
RES = 50
NVOX = RES * RES * RES
GRID_PAD = 125008

NC, NS, L = 2, 16, 16

BB = 8


def _enc_body(pts_ref, valid_ref, enc_ref):
    def coord(c):
        t = (pts_ref[c] + 2.0) * 0.25 * 49.0
        ti = jnp.floor(t).astype(jnp.int32)
        return jnp.clip(ti, 0, 49)

    i, j, k = coord(0), coord(1), coord(2)
    lin = (i * 50 + j) * 50 + k
    enc_ref[...] = jnp.where(valid_ref[...], lin, -1).reshape(BB * P)


def _encode(pts_t, valid):
    return pl.pallas_call(
        _enc_body,
        grid=(B // BB,),
        in_specs=[
            pl.BlockSpec((3, BB, P), lambda b: (0, b, 0)),
            pl.BlockSpec((BB, P), lambda b: (b, 0)),
        ],
        out_specs=pl.BlockSpec((BB * P,), lambda b: (b,)),
        out_shape=jax.ShapeDtypeStruct((B * P,), jnp.int32),
    )(pts_t, valid)


ECHUNK = 2048
NECH = P // ECHUNK
OCHUNK = 2048
NOFULL = NVOX // OCHUNK
OTAIL = NVOX - NOFULL * OCHUNK

_mesh = plsc.VectorSubcoreMesh(
    core_axis_name="c", subcore_axis_name="s", num_cores=NC, num_subcores=NS)


@functools.partial(
    pl.kernel,
    out_type=jax.ShapeDtypeStruct((B * NVOX,), jnp.float32),
    mesh=_mesh,
    compiler_params=pltpu.CompilerParams(needs_layout_passes=False),
    scratch_types=[
        pltpu.VMEM((GRID_PAD,), jnp.float32),
        pltpu.VMEM((2 * ECHUNK,), jnp.int32),
        pltpu.SemaphoreType.DMA,
        pltpu.SemaphoreType.DMA,
        pltpu.SemaphoreType.DMA,
    ],
)
def _voxelize(enc_hbm, out_hbm, grid_v, buf_v, insem0, insem1, outsem):
    cid = lax.axis_index("c")
    sid = lax.axis_index("s")
    wid = sid * NC + cid

    @pl.when(wid < B)
    def _work():
        b = wid
        zeros16 = jnp.zeros((L,), jnp.float32)
        ones16 = jnp.ones((L,), jnp.float32)
        insems = (insem0, insem1)

        def in_copy(c, bu):
            return pltpu.make_async_copy(
                enc_hbm.at[pl.ds(b * P + c * ECHUNK, ECHUNK)],
                buf_v.at[pl.ds(bu * ECHUNK, ECHUNK)],
                insems[bu])

        def out_copy(c, n):
            return pltpu.make_async_copy(
                grid_v.at[pl.ds(c * OCHUNK, n)],
                out_hbm.at[pl.ds(b * NVOX + c * OCHUNK, n)],
                outsem)

        in_copy(0, 0).start()
        in_copy(1, 1).start()

        def zbody(i, carry):
            grid_v[pl.ds(i * L, L)] = zeros16
            return carry
        lax.fori_loop(0, GRID_PAD // L, zbody, 0, unroll=8)

        def scpair(g, carry):
            for bu in (0, 1):
                c = g * 2 + bu
                in_copy(c, bu).wait()

                def vbody(v, carry2, bu=bu):
                    ev = buf_v[pl.ds(bu * ECHUNK + v * L, L)]
                    plsc.store_scatter(grid_v, [ev], ones16, mask=ev >= 0)
                    return carry2
                lax.fori_loop(0, ECHUNK // L, vbody, 0, unroll=8)

                @pl.when(c + 2 < NECH)
                def _(c=c, bu=bu):
                    in_copy(c + 2, bu).start()
            return carry
        lax.fori_loop(0, NECH // 2, scpair, 0)

        for c in range(NOFULL):
            out_copy(c, OCHUNK).start()
        out_copy(NOFULL, OTAIL).start()
        for c in range(NOFULL):
            out_copy(c, OCHUNK).wait()
        out_copy(NOFULL, OTAIL).wait()


def kernel(pointclouds, valid_points):
    pts_t = pointclouds.transpose(2, 0, 1)
    enc = _encode(pts_t, valid_points)
    occ = _voxelize(enc)
    occ5 = occ.reshape(B, RES, RES, RES, 1)
    return jnp.pad(occ5, ((0, 0), (0, 0), (0, 0), (0, 0), (0, 3)))
